# baseline (speedup 1.0000x reference)
import functools

import jax
import jax.numpy as jnp
from jax import lax
from jax.experimental import pallas as pl
from jax.experimental.pallas import tpu as pltpu
from jax.experimental.pallas import tpu_sc as plsc

_B = 16384
_D = 16
_NC = 2
_NS = 16
_NW = _NC * _NS
_BPW = _B // _NW
_CH = 128
_NCH = _BPW // _CH
_RB = 16


def _body(user_hbm, item_hbm, utab_hbm, itab_hbm, out_hbm,
          idx_u, idx_i, rows_u, rows_i, prod_v, out_v, sem):
    wid = lax.axis_index("s") * _NC + lax.axis_index("c")

    pltpu.sync_copy(user_hbm.at[pl.ds(wid * _NCH, _NCH)], idx_u)
    pltpu.sync_copy(item_hbm.at[pl.ds(wid * _NCH, _NCH)], idx_i)

    cps = []
    for c in range(_NCH):
        cps.append(pltpu.async_copy(utab_hbm.at[idx_u.at[c]],
                                    rows_u.at[pl.ds(c * _CH, _CH)], sem))
        cps.append(pltpu.async_copy(itab_hbm.at[idx_i.at[c]],
                                    rows_i.at[pl.ds(c * _CH, _CH)], sem))
    for cp in cps:
        cp.wait()

    lane = lax.iota(jnp.int32, _RB)
    lane16 = lane * _D

    def block(r, carry):
        row0 = r * _RB
        for i in range(_RB):
            p = rows_u[row0 + i, :] * rows_i[row0 + i, :]
            prod_v[pl.ds((row0 + i) * _D, _D)] = p
        acc = jnp.zeros((_RB,), jnp.float32)
        base = row0 * _D + lane16
        for j in range(_D):
            acc = acc + plsc.load_gather(prod_v, [base + j])
        s = 1.0 / (1.0 + jnp.exp(-acc))
        plsc.store_scatter(out_v, [row0 + lane], s)
        return carry

    lax.fori_loop(0, _BPW // _RB, block, 0)

    pltpu.sync_copy(out_v, out_hbm.at[pl.ds(wid * _BPW, _BPW)])


def kernel(user, item, user_table, item_table):
    user2d = user.astype(jnp.int32).reshape(_B // _CH, _CH)
    item2d = item.astype(jnp.int32).reshape(_B // _CH, _CH)
    mesh = plsc.VectorSubcoreMesh(core_axis_name="c", subcore_axis_name="s")
    f = functools.partial(
        pl.kernel,
        out_type=jax.ShapeDtypeStruct((_B,), jnp.float32),
        mesh=mesh,
        scratch_types=[
            pltpu.VMEM((_NCH, _CH), jnp.int32),
            pltpu.VMEM((_NCH, _CH), jnp.int32),
            pltpu.VMEM((_BPW, _D), jnp.float32),
            pltpu.VMEM((_BPW, _D), jnp.float32),
            pltpu.VMEM((_BPW * _D,), jnp.float32),
            pltpu.VMEM((_BPW,), jnp.float32),
            pltpu.SemaphoreType.DMA,
        ],
        compiler_params=pltpu.CompilerParams(
            needs_layout_passes=False, use_tc_tiling_on_sc=False),
    )(_body)
    return f(user2d, item2d, user_table, item_table)
